# baseline (speedup 1.0000x reference)
import functools

import jax
import jax.numpy as jnp
from jax import lax
from jax.experimental import pallas as pl
from jax.experimental.pallas import tpu as pltpu
from jax.experimental.pallas import tpu_sc as plsc

B = 4096
L = 50
D = 64
BL = B * L

_info = plsc.get_sparse_core_info()
NC, NS, LANES = _info.num_cores, _info.num_subcores, _info.num_lanes
NW = NC * NS
TOK_PER_W = BL // NW
CHUNK = 128
NCHUNK = TOK_PER_W // CHUNK


def _make_kernel():
    mesh = plsc.VectorSubcoreMesh(core_axis_name="c", subcore_axis_name="s")

    @functools.partial(
        pl.kernel,
        mesh=mesh,
        out_type=jax.ShapeDtypeStruct((BL, D), jnp.float32),
        compiler_params=pltpu.CompilerParams(use_tc_tiling_on_sc=False),
        scratch_types=[
            pltpu.VMEM((NCHUNK, CHUNK), jnp.int32),
            pltpu.VMEM((NCHUNK, CHUNK), jnp.float32),
            pltpu.VMEM((CHUNK, D), jnp.float32),
            pltpu.SemaphoreType.DMA,
        ],
    )
    def k(tok_hbm, mask_hbm, table_hbm, out_hbm, tok_v, mask_v, rows_v, sem):
        w = lax.axis_index("s") * NC + lax.axis_index("c")
        pltpu.sync_copy(tok_hbm.at[w], tok_v)
        pltpu.sync_copy(mask_hbm.at[w], mask_v)

        def chunk_body(c, carry):
            pltpu.async_copy(table_hbm.at[tok_v.at[c]], rows_v, sem).wait()

            for j16 in range(CHUNK // LANES):
                mk16 = mask_v[c, pl.ds(j16 * LANES, LANES)]
                for j in range(LANES):
                    t = j16 * LANES + j
                    m = lax.gather(
                        mk16,
                        jnp.full((LANES, 1), j, jnp.int32),
                        lax.GatherDimensionNumbers(
                            offset_dims=(), collapsed_slice_dims=(0,),
                            start_index_map=(0,)),
                        (1,),
                        mode=lax.GatherScatterMode.PROMISE_IN_BOUNDS)
                    for q in range(D // LANES):
                        sl = pl.ds(q * LANES, LANES)
                        rows_v[t, sl] = rows_v[t, sl] * m

            base = w * TOK_PER_W + c * CHUNK
            pltpu.sync_copy(rows_v, out_hbm.at[pl.ds(base, CHUNK), :])
            return carry

        lax.fori_loop(0, NCHUNK, chunk_body, 0)

    return k


_sc_kernel = _make_kernel()


def kernel(token_ids, seq_lens, glove_table):
    tok3d = token_ids.reshape(NW, NCHUNK, CHUNK).astype(jnp.int32)
    mask = (jnp.arange(L, dtype=jnp.int32)[None, :]
            < seq_lens.astype(jnp.int32)[:, None]).astype(jnp.float32)
    mask3d = mask.reshape(NW, NCHUNK, CHUNK)
    out = _sc_kernel(tok3d, mask3d, glove_table)
    return out.reshape(B, L, D)
